# Initial kernel scaffold; baseline (speedup 1.0000x reference)
import jax
import jax.numpy as jnp
from jax.experimental import pallas as pl


def kernel(x, raw_weight):
    raise NotImplementedError("write your pallas kernel here")


# speedup vs baseline: 1.1059x; 1.1059x over previous
import functools

import jax
import jax.numpy as jnp
from jax import lax
from jax.experimental import pallas as pl
from jax.experimental.pallas import tpu as pltpu
from jax.experimental.pallas import tpu_sc as plsc

EMBED_DIM = 32
IDX_ROW = 128
NUM_WORKERS = 32
CHUNK_ROWS = 10


@functools.partial(jax.jit, static_argnames=())
def _sc_embedding_gather(x2d, table):
    R = x2d.shape[0]
    rows_per_w = R // NUM_WORKERS
    n_chunks = rows_per_w // CHUNK_ROWS
    chunk = CHUNK_ROWS * IDX_ROW
    total = R * IDX_ROW

    mesh = plsc.VectorSubcoreMesh(core_axis_name="c", subcore_axis_name="s")

    @functools.partial(
        pl.kernel,
        out_type=jax.ShapeDtypeStruct((total, EMBED_DIM), jnp.float32),
        mesh=mesh,
        scratch_types=[
            pltpu.VMEM((rows_per_w, IDX_ROW), jnp.int32),
            pltpu.VMEM((chunk, EMBED_DIM), jnp.float32),
            pltpu.SemaphoreType.DMA,
        ],
        compiler_params=pltpu.CompilerParams(use_tc_tiling_on_sc=False),
    )
    def body(x_hbm, tab_hbm, out_hbm, idx_v, rows_v, gsem):
        wid = lax.axis_index("s") * 2 + lax.axis_index("c")
        idx_base = wid * rows_per_w
        out_base = idx_base * IDX_ROW
        pltpu.sync_copy(x_hbm.at[pl.ds(idx_base, rows_per_w)], idx_v)

        def chunk_body(c, carry):
            copies = []
            for s in range(CHUNK_ROWS):
                cp = pltpu.make_async_copy(
                    tab_hbm.at[idx_v.at[c * CHUNK_ROWS + s]],
                    rows_v.at[pl.ds(s * IDX_ROW, IDX_ROW)],
                    gsem,
                )
                cp.start()
                copies.append(cp)
            for cp in copies:
                cp.wait()
            pltpu.sync_copy(rows_v, out_hbm.at[pl.ds(out_base + c * chunk, chunk)])
            return carry

        lax.fori_loop(0, n_chunks, chunk_body, 0)

    return body(x2d, table)


def kernel(x, raw_weight):
    B, S = x.shape
    x2d = x.reshape(-1, IDX_ROW).astype(jnp.int32)
    out = _sc_embedding_gather(x2d, raw_weight)
    return out.reshape(B, S, EMBED_DIM)


# speedup vs baseline: 1.1111x; 1.0047x over previous
import functools

import jax
import jax.numpy as jnp
from jax import lax
from jax.experimental import pallas as pl
from jax.experimental.pallas import tpu as pltpu
from jax.experimental.pallas import tpu_sc as plsc

EMBED_DIM = 32
IDX_ROW = 128
NUM_WORKERS = 32
CHUNK_ROWS = 10


@functools.partial(jax.jit, static_argnames=())
def _sc_embedding_gather(x2d, table):
    R = x2d.shape[0]
    rows_per_w = R // NUM_WORKERS
    n_chunks = rows_per_w // CHUNK_ROWS
    chunk = CHUNK_ROWS * IDX_ROW
    total = R * IDX_ROW

    mesh = plsc.VectorSubcoreMesh(core_axis_name="c", subcore_axis_name="s")

    @functools.partial(
        pl.kernel,
        out_type=jax.ShapeDtypeStruct((total, EMBED_DIM), jnp.float32),
        mesh=mesh,
        scratch_types=[
            pltpu.VMEM((rows_per_w, IDX_ROW), jnp.int32),
            pltpu.VMEM((2, chunk, EMBED_DIM), jnp.float32),
            pltpu.SemaphoreType.DMA,
            pltpu.SemaphoreType.DMA,
        ],
        compiler_params=pltpu.CompilerParams(use_tc_tiling_on_sc=False),
    )
    def body(x_hbm, tab_hbm, out_hbm, idx_v, rows_v, gsem, ssem):
        wid = lax.axis_index("s") * 2 + lax.axis_index("c")
        idx_base = wid * rows_per_w
        out_base = idx_base * IDX_ROW
        pltpu.sync_copy(x_hbm.at[pl.ds(idx_base, rows_per_w)], idx_v)

        def fire_gathers(c, slot):
            for s in range(CHUNK_ROWS):
                pltpu.make_async_copy(
                    tab_hbm.at[idx_v.at[c * CHUNK_ROWS + s]],
                    rows_v.at[slot, pl.ds(s * IDX_ROW, IDX_ROW)],
                    gsem,
                ).start()

        def wait_gathers(slot):
            pltpu.make_async_copy(
                tab_hbm.at[pl.ds(0, chunk)],
                rows_v.at[slot],
                gsem,
            ).wait()

        def store_desc(c, slot):
            return pltpu.make_async_copy(
                rows_v.at[slot],
                out_hbm.at[pl.ds(out_base + c * chunk, chunk)],
                ssem,
            )

        fire_gathers(0, 0)

        def chunk_body(c, carry):
            slot = lax.rem(c, 2)
            wait_gathers(slot)

            @pl.when(c >= 1)
            def _():
                store_desc(c - 1, 1 - slot).wait()

            @pl.when(c + 1 < n_chunks)
            def _():
                fire_gathers(c + 1, 1 - slot)

            store_desc(c, slot).start()
            return carry

        lax.fori_loop(0, n_chunks, chunk_body, 0)
        store_desc(n_chunks - 1, lax.rem(n_chunks - 1, 2)).wait()

    return body(x2d, table)


def kernel(x, raw_weight):
    B, S = x.shape
    x2d = x.reshape(-1, IDX_ROW).astype(jnp.int32)
    out = _sc_embedding_gather(x2d, raw_weight)
    return out.reshape(B, S, EMBED_DIM)


# speedup vs baseline: 1.7966x; 1.6170x over previous
import functools

import jax
import jax.numpy as jnp
from jax import lax
from jax.experimental import pallas as pl
from jax.experimental.pallas import tpu as pltpu
from jax.experimental.pallas import tpu_sc as plsc

EMBED_DIM = 32
NUM_WORKERS = 32
CHUNK_B = 16


def _sc_embedding_gather(x, table):
    B, S = x.shape
    b_per_w = B // NUM_WORKERS
    n_chunks = b_per_w // CHUNK_B

    mesh = plsc.VectorSubcoreMesh(core_axis_name="c", subcore_axis_name="s")

    @functools.partial(
        pl.kernel,
        out_type=jax.ShapeDtypeStruct((B, S, EMBED_DIM), jnp.float32),
        mesh=mesh,
        scratch_types=[
            pltpu.VMEM((b_per_w, S), jnp.int32),
            pltpu.VMEM((2, CHUNK_B, S, EMBED_DIM), jnp.float32),
            pltpu.SemaphoreType.DMA,
            pltpu.SemaphoreType.DMA,
        ],
        compiler_params=pltpu.CompilerParams(use_tc_tiling_on_sc=False),
    )
    def body(x_hbm, tab_hbm, out_hbm, idx_v, rows_v, gsem, ssem):
        wid = lax.axis_index("s") * 2 + lax.axis_index("c")
        base = wid * b_per_w
        pltpu.sync_copy(x_hbm.at[pl.ds(base, b_per_w)], idx_v)

        def fire_gather(c, slot):
            for r in range(CHUNK_B):
                pltpu.make_async_copy(
                    tab_hbm.at[idx_v.at[c * CHUNK_B + r]],
                    rows_v.at[slot, r],
                    gsem,
                ).start()

        def wait_gather(c, slot):
            for r in range(CHUNK_B):
                pltpu.make_async_copy(
                    tab_hbm.at[idx_v.at[c * CHUNK_B + r]],
                    rows_v.at[slot, r],
                    gsem,
                ).wait()

        def store_desc(c, slot):
            return pltpu.make_async_copy(
                rows_v.at[slot],
                out_hbm.at[pl.ds(base + c * CHUNK_B, CHUNK_B)],
                ssem,
            )

        fire_gather(0, 0)

        def chunk_body(c, carry):
            slot = lax.rem(c, 2)
            wait_gather(c, slot)

            @pl.when(c >= 1)
            def _():
                store_desc(c - 1, 1 - slot).wait()

            @pl.when(c + 1 < n_chunks)
            def _():
                fire_gather(c + 1, 1 - slot)

            store_desc(c, slot).start()
            return carry

        lax.fori_loop(0, n_chunks, chunk_body, 0)
        store_desc(n_chunks - 1, lax.rem(n_chunks - 1, 2)).wait()

    return body(x, table)


def kernel(x, raw_weight):
    return _sc_embedding_gather(x.astype(jnp.int32), raw_weight)
